# baseline (speedup 1.0000x reference)
import functools

import jax
import jax.numpy as jnp
from jax import lax
from jax.experimental import pallas as pl
from jax.experimental.pallas import tpu as pltpu
from jax.experimental.pallas import tpu_sc as plsc

NUM_CORES = 2
NUM_SUBCORES = 16
NUM_WORKERS = NUM_CORES * NUM_SUBCORES
LANES = 16
BATCH = 16384
DIM = 32
BPW = BATCH // NUM_WORKERS
RING = 8
KB = BPW // 128


def _gmf_body(uidx_hbm, iidx_hbm, utabT_hbm, itabT_hbm, wb_hbm, out_hbm,
              uidx_v, iidx_v, ublk_v, iblk_v, uT_v, iT_v,
              wb_v, out_v, sem_u, sem_i):
    c = lax.axis_index("c")
    s = lax.axis_index("s")
    wid = s * NUM_CORES + c
    base = pl.multiple_of(wid * BPW, BPW)

    pltpu.sync_copy(wb_hbm, wb_v)
    pltpu.sync_copy(uidx_hbm.at[pl.ds(base, BPW)], uidx_v.at[pl.ds(0, BPW)])
    pltpu.sync_copy(iidx_hbm.at[pl.ds(base, BPW)], iidx_v.at[pl.ds(0, BPW)])
    uidx_v[pl.ds(BPW, LANES)] = jnp.zeros((LANES,), jnp.int32)
    iidx_v[pl.ds(BPW, LANES)] = jnp.zeros((LANES,), jnp.int32)

    lanes16 = lax.iota(jnp.int32, LANES)

    def issue(r_u, r_i, slot):
        ub = pl.multiple_of((r_u // 128) * 128, 128)
        ib = pl.multiple_of((r_i // 128) * 128, 128)
        for dh in range(4):
            pltpu.async_copy(
                utabT_hbm.at[pl.ds(dh * 8, 8), pl.ds(ub, 128)],
                ublk_v.at[slot, pl.ds(dh * 8, 8)], sem_u.at[slot])
            pltpu.async_copy(
                itabT_hbm.at[pl.ds(dh * 8, 8), pl.ds(ib, 128)],
                iblk_v.at[slot, pl.ds(dh * 8, 8)], sem_i.at[slot])

    def extract(blk_v, dst_v, col, k, sl):
        kb = k // 128
        ko = lax.rem(k, 128)
        for half in range(2):
            src_idx = [jnp.full((LANES,), sl, jnp.int32),
                       lanes16 + (half * LANES),
                       jnp.full((LANES,), col, jnp.int32)]
            vals = plsc.load_gather(blk_v, src_idx)
            dst_idx = [lanes16 * KB + (half * LANES * KB + kb),
                       jnp.full((LANES,), ko, jnp.int32)]
            plsc.store_scatter(dst_v, dst_idx, vals)

    def collect(r_u, r_i, k, slot):
        for dh in range(4):
            pltpu.make_async_copy(
                utabT_hbm.at[pl.ds(dh * 8, 8), pl.ds(0, 128)],
                ublk_v.at[slot, pl.ds(dh * 8, 8)], sem_u.at[slot]).wait()
            pltpu.make_async_copy(
                itabT_hbm.at[pl.ds(dh * 8, 8), pl.ds(0, 128)],
                iblk_v.at[slot, pl.ds(dh * 8, 8)], sem_i.at[slot]).wait()
        extract(ublk_v, uT_v, lax.rem(r_u, 128), k, slot)
        extract(iblk_v, iT_v, lax.rem(r_i, 128), k, slot)

    NBLK = BPW // RING

    def prime_body(_, carry):
        uvec = uidx_v[pl.ds(0, LANES)]
        ivec = iidx_v[pl.ds(0, LANES)]
        for j in range(RING):
            issue(uvec[j], ivec[j], j)
        return carry

    def steady_body(b, carry):
        off = pl.multiple_of((b - 1) * RING, RING)
        uvec = uidx_v[pl.ds(off, LANES)]
        ivec = iidx_v[pl.ds(off, LANES)]
        for j in range(RING):
            collect(uvec[j], ivec[j], off + j, j)
            issue(uvec[RING + j], ivec[RING + j], j)
        return carry

    def drain_body(_, carry):
        off = BPW - RING
        uvec = uidx_v[pl.ds(off, LANES)]
        ivec = iidx_v[pl.ds(off, LANES)]
        for j in range(RING):
            collect(uvec[j], ivec[j], off + j, j)
        return carry

    lax.fori_loop(0, 1, prime_body, 0)
    lax.fori_loop(1, NBLK, steady_body, 0)
    lax.fori_loop(0, 1, drain_body, 0)

    bias_v = wb_v[DIM, :]

    def group_body(g, carry):
        kb = g // (128 // LANES)
        ko = pl.multiple_of(lax.rem(g, 128 // LANES) * LANES, LANES)
        acc = bias_v
        for d in range(DIM):
            uv = uT_v[d * KB + kb, pl.ds(ko, LANES)]
            iv = iT_v[d * KB + kb, pl.ds(ko, LANES)]
            wv = wb_v[d, :]
            acc = acc + uv * iv * wv
        out_v[pl.ds(pl.multiple_of(g * LANES, LANES), LANES)] = (
            1.0 / (1.0 + jnp.exp(-acc)))
        return carry

    lax.fori_loop(0, BPW // LANES, group_body, 0)
    pltpu.sync_copy(out_v, out_hbm.at[pl.ds(base, BPW)])


@jax.jit
def _gmf_call(ui, ii, utabT, itabT, wb):
    mesh = plsc.VectorSubcoreMesh(core_axis_name="c", subcore_axis_name="s")
    f = functools.partial(
        pl.kernel,
        out_type=jax.ShapeDtypeStruct((BATCH,), jnp.float32),
        mesh=mesh,
        compiler_params=pltpu.CompilerParams(needs_layout_passes=False),
        scratch_types=[
            pltpu.VMEM((BPW + LANES,), jnp.int32),
            pltpu.VMEM((BPW + LANES,), jnp.int32),
            pltpu.VMEM((RING, DIM, 128), jnp.float32),
            pltpu.VMEM((RING, DIM, 128), jnp.float32),
            pltpu.VMEM((DIM * KB, 128), jnp.float32),
            pltpu.VMEM((DIM * KB, 128), jnp.float32),
            pltpu.VMEM((DIM + 1, LANES), jnp.float32),
            pltpu.VMEM((BPW,), jnp.float32),
            pltpu.SemaphoreType.DMA((RING,)),
            pltpu.SemaphoreType.DMA((RING,)),
        ],
    )(_gmf_body)
    return f(ui, ii, utabT, itabT, wb)


def kernel(user_indices, item_indices, user_table, item_table, affine_w, affine_b):
    ui = user_indices.astype(jnp.int32)
    ii = item_indices.astype(jnp.int32)
    wb = jnp.concatenate([
        jnp.broadcast_to(affine_w.reshape(DIM, 1), (DIM, LANES)),
        jnp.broadcast_to(affine_b.reshape(1, 1), (1, LANES)),
    ], axis=0).astype(jnp.float32)
    out = _gmf_call(ui, ii, user_table.T, item_table.T, wb)
    return out.reshape(BATCH, 1)


# speedup vs baseline: 1.0068x; 1.0068x over previous
import functools

import jax
import jax.numpy as jnp
from jax import lax
from jax.experimental import pallas as pl
from jax.experimental.pallas import tpu as pltpu
from jax.experimental.pallas import tpu_sc as plsc

NUM_CORES = 2
NUM_SUBCORES = 16
NUM_WORKERS = NUM_CORES * NUM_SUBCORES
LANES = 16
BATCH = 16384
DIM = 32
BPW = BATCH // NUM_WORKERS
RING = 8
KB = BPW // 128


def _gmf_body(uidx_hbm, iidx_hbm, utabT_hbm, itabT_hbm, wb_hbm, out_hbm,
              uidx_v, iidx_v, ublk_v, iblk_v, uT_v, iT_v,
              wb_v, out_v, sem_u, sem_i):
    c = lax.axis_index("c")
    s = lax.axis_index("s")
    wid = s * NUM_CORES + c
    base = pl.multiple_of(wid * BPW, BPW)

    pltpu.sync_copy(wb_hbm, wb_v)
    pltpu.sync_copy(uidx_hbm.at[pl.ds(base, BPW)], uidx_v.at[pl.ds(0, BPW)])
    pltpu.sync_copy(iidx_hbm.at[pl.ds(base, BPW)], iidx_v.at[pl.ds(0, BPW)])
    uidx_v[pl.ds(BPW, LANES)] = jnp.zeros((LANES,), jnp.int32)
    iidx_v[pl.ds(BPW, LANES)] = jnp.zeros((LANES,), jnp.int32)

    lanes16 = lax.iota(jnp.int32, LANES)

    def issue(r_u, r_i, slot):
        ub = pl.multiple_of((r_u // 128) * 128, 128)
        ib = pl.multiple_of((r_i // 128) * 128, 128)
        pltpu.async_copy(utabT_hbm.at[:, pl.ds(ub, 128)], ublk_v.at[slot],
                         sem_u.at[slot])
        pltpu.async_copy(itabT_hbm.at[:, pl.ds(ib, 128)], iblk_v.at[slot],
                         sem_i.at[slot])

    def extract(blk_v, dst_v, col, k, sl):
        kb = k // 128
        ko = lax.rem(k, 128)
        for half in range(2):
            src_idx = [jnp.full((LANES,), sl, jnp.int32),
                       lanes16 + (half * LANES),
                       jnp.full((LANES,), col, jnp.int32)]
            vals = plsc.load_gather(blk_v, src_idx)
            dst_idx = [lanes16 * KB + (half * LANES * KB + kb),
                       jnp.full((LANES,), ko, jnp.int32)]
            plsc.store_scatter(dst_v, dst_idx, vals)

    def collect(r_u, r_i, k, slot):
        pltpu.make_async_copy(utabT_hbm.at[:, pl.ds(0, 128)],
                              ublk_v.at[slot], sem_u.at[slot]).wait()
        pltpu.make_async_copy(itabT_hbm.at[:, pl.ds(0, 128)],
                              iblk_v.at[slot], sem_i.at[slot]).wait()
        extract(ublk_v, uT_v, lax.rem(r_u, 128), k, slot)
        extract(iblk_v, iT_v, lax.rem(r_i, 128), k, slot)

    NBLK = BPW // RING

    bias_v = wb_v[DIM, :]

    def group_body(g, carry):
        kb = g // (128 // LANES)
        ko = pl.multiple_of(lax.rem(g, 128 // LANES) * LANES, LANES)
        acc = bias_v
        for d in range(DIM):
            uv = uT_v[d * KB + kb, pl.ds(ko, LANES)]
            iv = iT_v[d * KB + kb, pl.ds(ko, LANES)]
            wv = wb_v[d, :]
            acc = acc + uv * iv * wv
        out_v[pl.ds(pl.multiple_of(g * LANES, LANES), LANES)] = (
            1.0 / (1.0 + jnp.exp(-acc)))
        return carry

    def prime_body(_, carry):
        uvec = uidx_v[pl.ds(0, LANES)]
        ivec = iidx_v[pl.ds(0, LANES)]
        for j in range(RING):
            issue(uvec[j], ivec[j], j)
        return carry

    def steady_body(b, carry):
        off = pl.multiple_of((b - 1) * RING, RING)
        uvec = uidx_v[pl.ds(off, LANES)]
        ivec = iidx_v[pl.ds(off, LANES)]
        for j in range(RING):
            collect(uvec[j], ivec[j], off + j, j)
            issue(uvec[RING + j], ivec[RING + j], j)
        @pl.when(lax.rem(b, 2) == 0)
        def _():
            group_body(b // 2 - 1, 0)
        return carry

    def drain_body(_, carry):
        off = BPW - RING
        uvec = uidx_v[pl.ds(off, LANES)]
        ivec = iidx_v[pl.ds(off, LANES)]
        for j in range(RING):
            collect(uvec[j], ivec[j], off + j, j)
        return carry

    lax.fori_loop(0, 1, prime_body, 0)
    lax.fori_loop(1, NBLK, steady_body, 0)
    lax.fori_loop(0, 1, drain_body, 0)

    lax.fori_loop(BPW // LANES - 1, BPW // LANES, group_body, 0)
    pltpu.sync_copy(out_v, out_hbm.at[pl.ds(base, BPW)])


@jax.jit
def _gmf_call(ui, ii, utabT, itabT, wb):
    mesh = plsc.VectorSubcoreMesh(core_axis_name="c", subcore_axis_name="s")
    f = functools.partial(
        pl.kernel,
        out_type=jax.ShapeDtypeStruct((BATCH,), jnp.float32),
        mesh=mesh,
        compiler_params=pltpu.CompilerParams(needs_layout_passes=False),
        scratch_types=[
            pltpu.VMEM((BPW + LANES,), jnp.int32),
            pltpu.VMEM((BPW + LANES,), jnp.int32),
            pltpu.VMEM((RING, DIM, 128), jnp.float32),
            pltpu.VMEM((RING, DIM, 128), jnp.float32),
            pltpu.VMEM((DIM * KB, 128), jnp.float32),
            pltpu.VMEM((DIM * KB, 128), jnp.float32),
            pltpu.VMEM((DIM + 1, LANES), jnp.float32),
            pltpu.VMEM((BPW,), jnp.float32),
            pltpu.SemaphoreType.DMA((RING,)),
            pltpu.SemaphoreType.DMA((RING,)),
        ],
    )(_gmf_body)
    return f(ui, ii, utabT, itabT, wb)


def kernel(user_indices, item_indices, user_table, item_table, affine_w, affine_b):
    ui = user_indices.astype(jnp.int32)
    ii = item_indices.astype(jnp.int32)
    wb = jnp.concatenate([
        jnp.broadcast_to(affine_w.reshape(DIM, 1), (DIM, LANES)),
        jnp.broadcast_to(affine_b.reshape(1, 1), (1, LANES)),
    ], axis=0).astype(jnp.float32)
    out = _gmf_call(ui, ii, user_table.T, item_table.T, wb)
    return out.reshape(BATCH, 1)
